# baseline (speedup 1.0000x reference)
import functools

import jax
import jax.numpy as jnp
from jax import lax
from jax.experimental import pallas as pl
from jax.experimental.pallas import tpu as pltpu
from jax.experimental.pallas import tpu_sc as plsc

N_USERS = 1000000
N_ITEMS = 1000000
EMB_DIM = 64
BATCH = 16384

_INFO = plsc.get_sparse_core_info()
_NC = _INFO.num_cores
_NS = _INFO.num_subcores
_NW = _NC * _NS
_B_PER_W = BATCH // _NW
_CHUNK = 128
_NCHUNK = _B_PER_W // _CHUNK
_GROUPS_PER_CHUNK = _CHUNK // 16


def _mf_body(u_hbm, i_hbm, ut_hbm, it_hbm, out_hbm,
             idx_u, idx_i, rows_u, rows_i, out_v,
             sem_u0, sem_u1, sem_u2, sem_u3,
             sem_i0, sem_i1, sem_i2, sem_i3):
    wid = lax.axis_index("s") * _NC + lax.axis_index("c")
    base = wid * _B_PER_W

    sems_u = (sem_u0, sem_u1, sem_u2, sem_u3)
    sems_i = (sem_i0, sem_i1, sem_i2, sem_i3)

    for j in range(_NCHUNK):
        pltpu.sync_copy(u_hbm.at[pl.ds(base + j * _CHUNK, _CHUNK)], idx_u.at[j])
        pltpu.sync_copy(i_hbm.at[pl.ds(base + j * _CHUNK, _CHUNK)], idx_i.at[j])

    cps_u = [
        pltpu.async_copy(ut_hbm.at[idx_u.at[j]],
                         rows_u.at[pl.ds(j * _CHUNK, _CHUNK)], sems_u[j])
        for j in range(_NCHUNK)
    ]
    cps_i = [
        pltpu.async_copy(it_hbm.at[idx_i.at[j]],
                         rows_i.at[pl.ds(j * _CHUNK, _CHUNK)], sems_i[j])
        for j in range(_NCHUNK)
    ]

    lane = lax.iota(jnp.int32, 16)

    for j in range(_NCHUNK):
        cps_u[j].wait()
        cps_i[j].wait()

        def group(g, carry, j=j):
            row0 = j * _CHUNK + g * 16
            rows16 = row0 + lane
            acc = jnp.zeros((16,), jnp.float32)
            for k in range(EMB_DIM):
                kk = jnp.full((16,), k, jnp.int32)
                uv = plsc.load_gather(rows_u, [rows16, kk])
                iv = plsc.load_gather(rows_i, [rows16, kk])
                acc = acc + uv * iv
            out_v[pl.ds(row0, 16)] = acc
            return carry

        lax.fori_loop(0, _GROUPS_PER_CHUNK, group, 0)

    pltpu.sync_copy(out_v, out_hbm.at[pl.ds(base, _B_PER_W)])


@jax.jit
def _mf_sc(u, i, user_table, item_table):
    mesh = plsc.VectorSubcoreMesh(core_axis_name="c", subcore_axis_name="s")
    f = pl.kernel(
        _mf_body,
        mesh=mesh,
        out_type=jax.ShapeDtypeStruct((BATCH,), jnp.float32),
        scratch_types=[
            pltpu.VMEM((_NCHUNK, _CHUNK), jnp.int32),
            pltpu.VMEM((_NCHUNK, _CHUNK), jnp.int32),
            pltpu.VMEM((_B_PER_W, EMB_DIM), jnp.float32),
            pltpu.VMEM((_B_PER_W, EMB_DIM), jnp.float32),
            pltpu.VMEM((_B_PER_W,), jnp.float32),
        ] + [pltpu.SemaphoreType.DMA] * (2 * _NCHUNK),
        compiler_params=pltpu.CompilerParams(
            needs_layout_passes=False, use_tc_tiling_on_sc=False),
    )
    return f(u, i, user_table, item_table)


def kernel(u, i, user_table, item_table):
    return _mf_sc(u, i, user_table, item_table)
